# baseline (speedup 1.0000x reference)
import functools

import jax
import jax.numpy as jnp
from jax.experimental import pallas as pl
from jax.experimental.pallas import tpu as pltpu

D_MODEL = 4096
NUM_EXPERTS = 64
TOKENS = 4 * 2048
CHUNK_T = 128
CPS = 4
BLOCK_T = CHUNK_T * CPS
GRID = TOKENS // BLOCK_T
NBUF = 16
AHEAD = 3


def _fetch(x_hbm, xbuf, sem, chunk, slot):
    return pltpu.make_async_copy(
        x_hbm.at[pl.ds(chunk * CHUNK_T, CHUNK_T), :],
        xbuf.at[slot],
        sem.at[slot],
    )


def _router_body(x_hbm, wt_ref, b_ref, disp_ref, counts_ref, loss_ref,
                 xbuf, sem):
    step = pl.program_id(0)

    @pl.when(step == 0)
    def _():
        for c in range(CPS * AHEAD):
            _fetch(x_hbm, xbuf, sem, jnp.int32(c), jnp.int32(c % NBUF)).start()

    nxt_step = step + AHEAD

    @pl.when(nxt_step < GRID)
    def _():
        for j in range(CPS):
            c = nxt_step * CPS + j
            _fetch(x_hbm, xbuf, sem, c, jax.lax.rem(c, NBUF)).start()

    base = jax.lax.rem(step * CPS, NBUF)
    partial = jnp.zeros((1, NUM_EXPERTS), jnp.float32)
    lanes = jax.lax.broadcasted_iota(jnp.int32, (CHUNK_T, NUM_EXPERTS), 1)
    for j in range(CPS):
        slot = base + j
        _fetch(x_hbm, xbuf, sem, step * CPS + j, slot).wait()
        logits = jnp.dot(xbuf[slot], wt_ref[...],
                         preferred_element_type=jnp.float32)
        logits = logits + b_ref[...]
        idx = jnp.argmax(logits, axis=1)
        onehot = (lanes == idx[:, None]).astype(jnp.float32)
        disp_ref[pl.ds(j * CHUNK_T, CHUNK_T), :] = onehot
        partial = partial + jnp.sum(onehot, axis=0, keepdims=True)

    @pl.when(step == 0)
    def _():
        counts_ref[...] = partial

    @pl.when(step > 0)
    def _():
        counts_ref[...] = counts_ref[...] + partial

    @pl.when(step == GRID - 1)
    def _():
        counts = counts_ref[...]
        total = jnp.maximum(jnp.sum(counts), 1.0)
        lb = counts * (NUM_EXPERTS / total)
        loss_ref[...] = jnp.mean((lb - 1.0) ** 2).reshape(1, 1)


@functools.partial(jax.jit, static_argnames=())
def kernel(x, W, b):
    xf = x.reshape(TOKENS, D_MODEL)
    wt = W.T
    b2 = b.reshape(1, NUM_EXPERTS)
    disp, counts, loss = pl.pallas_call(
        _router_body,
        grid=(GRID,),
        in_specs=[
            pl.BlockSpec(memory_space=pltpu.MemorySpace.HBM),
            pl.BlockSpec((D_MODEL, NUM_EXPERTS), lambda i: (0, 0)),
            pl.BlockSpec((1, NUM_EXPERTS), lambda i: (0, 0)),
        ],
        out_specs=[
            pl.BlockSpec((BLOCK_T, NUM_EXPERTS), lambda i: (i, 0)),
            pl.BlockSpec((1, NUM_EXPERTS), lambda i: (0, 0)),
            pl.BlockSpec((1, 1), lambda i: (0, 0)),
        ],
        out_shape=[
            jax.ShapeDtypeStruct((TOKENS, NUM_EXPERTS), jnp.float32),
            jax.ShapeDtypeStruct((1, NUM_EXPERTS), jnp.float32),
            jax.ShapeDtypeStruct((1, 1), jnp.float32),
        ],
        scratch_shapes=[
            pltpu.VMEM((NBUF, CHUNK_T, D_MODEL), jnp.float32),
            pltpu.SemaphoreType.DMA((NBUF,)),
        ],
    )(xf, wt, b2)
    dispatch = disp.reshape(x.shape[0], x.shape[1], NUM_EXPERTS)
    expert_counts = counts.reshape(NUM_EXPERTS)
    load_balance_loss = loss[0, 0]
    return dispatch, dispatch, expert_counts, load_balance_loss, expert_counts
